# baseline (speedup 1.0000x reference)
import jax
import jax.numpy as jnp
from jax.experimental import pallas as pl

_B, _L, _C = 4096, 20, 1000
_N = _B * _L
_ROWS = 1024


def _onehot_block(x_ref, o_ref):
    xb = x_ref[...]
    act = xb[:, 0:1].astype(jnp.int32)
    dur = xb[:, 1:2]
    col = jax.lax.broadcasted_iota(jnp.int32, (_ROWS, _C + 1), 1)
    o_ref[...] = (col == act).astype(jnp.float32)
    o_ref[:, _C:_C + 1] = dur


def kernel(x):
    xf = x.reshape(_N, 2)
    out = pl.pallas_call(
        _onehot_block,
        grid=(_N // _ROWS,),
        in_specs=[pl.BlockSpec((_ROWS, 2), lambda i: (i, 0))],
        out_specs=pl.BlockSpec((_ROWS, _C + 1), lambda i: (i, 0)),
        out_shape=jax.ShapeDtypeStruct((_N, _C + 1), jnp.float32),
    )(xf)
    return out.reshape(_B, _L, _C + 1)
